# Initial kernel scaffold; baseline (speedup 1.0000x reference)
import jax
import jax.numpy as jnp
from jax.experimental import pallas as pl


def kernel(x, adj_t, w1_l, b1_l, w1_r, w2_l, b2_l, w2_r):
    raise NotImplementedError("write your pallas kernel here")


# speedup vs baseline: 1.1340x; 1.1340x over previous
import functools

import jax
import jax.numpy as jnp
from jax import lax
from jax.experimental import pallas as pl
from jax.experimental.pallas import tpu as pltpu

NEG = -1e30
TILE = 128
SLAB = 8


def _layer_kernel(bits_ref, adj_ref, x_ref, wl_ref, wr_ref, b_ref, out_ref,
                  madd_ref, agg_ref, *, apply_relu):
    t = pl.program_id(0)
    n_src = adj_ref.shape[0]

    madd_ref[...] = (adj_ref[...].astype(jnp.float32) - 1.0) * -NEG
    agg_ref[...] = jnp.full(agg_ref.shape, NEG, agg_ref.dtype)

    def slab_body(s, carry):
        w = bits_ref[t, s]

        @pl.when(w != 0)
        def _slab():
            base = pl.multiple_of(s * SLAB, SLAB)
            mt = jnp.transpose(madd_ref[pl.ds(base, SLAB), :])
            xs = x_ref[pl.ds(base, SLAB), :]
            for r in range(SLAB):
                @pl.when(((w >> r) & 1) != 0)
                def _row():
                    cand = mt[:, r:r + 1] + xs[r:r + 1, :]
                    agg_ref[...] = jnp.maximum(agg_ref[...], cand)

        return carry

    lax.fori_loop(0, n_src // SLAB, slab_body, 0)

    agg = agg_ref[...]
    agg = jnp.where(agg < NEG * 0.5, 0.0, agg)
    xt = x_ref[pl.ds(pl.multiple_of(t * TILE, TILE), TILE), :]
    out = (jnp.dot(agg, wl_ref[...], preferred_element_type=jnp.float32)
           + jnp.dot(xt, wr_ref[...], preferred_element_type=jnp.float32)
           + b_ref[...])
    if apply_relu:
        out = jnp.maximum(out, 0.0)
    out_ref[...] = out


def _sage_layer(bits, adj_i8, x, w_l, b_l, w_r, *, apply_relu):
    n, f = x.shape
    h = w_l.shape[1]
    kern = functools.partial(_layer_kernel, apply_relu=apply_relu)
    return pl.pallas_call(
        kern,
        out_shape=jax.ShapeDtypeStruct((n, h), jnp.float32),
        grid=(n // TILE,),
        in_specs=[
            pl.BlockSpec(memory_space=pltpu.SMEM),
            pl.BlockSpec((n, TILE), lambda t: (0, t)),
            pl.BlockSpec((n, f), lambda t: (0, 0)),
            pl.BlockSpec((f, h), lambda t: (0, 0)),
            pl.BlockSpec((f, h), lambda t: (0, 0)),
            pl.BlockSpec((1, h), lambda t: (0, 0)),
        ],
        out_specs=pl.BlockSpec((TILE, h), lambda t: (t, 0)),
        scratch_shapes=[
            pltpu.VMEM((n, TILE), jnp.float32),
            pltpu.VMEM((TILE, f), jnp.float32),
        ],
        compiler_params=pltpu.CompilerParams(
            dimension_semantics=("parallel",)),
    )(bits, adj_i8, x, w_l, w_r, b_l)


def kernel(x, adj_t, w1_l, b1_l, w1_r, w2_l, b2_l, w2_r):
    n = x.shape[0]
    edge = adj_t != 0
    adj_i8 = edge.astype(jnp.int8)
    rowany = edge.reshape(n, n // TILE, TILE).any(axis=-1)
    pow2 = (2 ** jnp.arange(SLAB)).astype(jnp.float32)
    bits = (rowany.T.astype(jnp.float32)
            .reshape(n // TILE, n // SLAB, SLAB) * pow2).sum(-1).astype(jnp.int32)

    x = x.astype(jnp.float32)
    b1 = jnp.reshape(b1_l, (1, -1)).astype(jnp.float32)
    b2 = jnp.reshape(b2_l, (1, -1)).astype(jnp.float32)

    h = _sage_layer(bits, adj_i8, x, w1_l, b1, w1_r, apply_relu=True)
    out = _sage_layer(bits, adj_i8, h, w2_l, b2, w2_r, apply_relu=False)
    return out
